# Initial kernel scaffold; baseline (speedup 1.0000x reference)
import jax
import jax.numpy as jnp
from jax.experimental import pallas as pl


def kernel(x, y, Wq, Wkv, Wproj, bproj, ln_w, ln_b, a1, a2):
    raise NotImplementedError("write your pallas kernel here")


# speedup vs baseline: 44.6076x; 44.6076x over previous
import functools

import jax
import jax.numpy as jnp
import numpy as np
from jax.experimental import pallas as pl

NUM_HEADS = 8
_SIGN = np.int32(-2**31)


def _pool_matrix(hw: int) -> np.ndarray:
    p = np.zeros((hw * hw, hw * hw), np.float32)
    idx = np.arange(hw)
    for k in (3, 5, 7):
        a = (np.abs(idx[:, None] - idx[None, :]) <= k // 2).astype(np.float32)
        p += np.kron(a, a) / float(k * k)
    return p


def _lnqkv_body(yT_ref, xf_ref, pm_ref, wkv_ref, wq_ref, lnw_ref, lnb_ref,
                kv_ref, q_ref):
    ys = jnp.dot(pm_ref[...], yT_ref[0],
                 preferred_element_type=jnp.float32,
                 precision=jax.lax.Precision.HIGHEST)
    mu = jnp.mean(ys, axis=-1, keepdims=True)
    var = jnp.mean((ys - mu) ** 2, axis=-1, keepdims=True)
    yn = (ys - mu) / jnp.sqrt(var + 1e-5) * lnw_ref[...] + lnb_ref[...]
    kv_ref[0] = jnp.dot(yn.astype(jnp.bfloat16),
                        wkv_ref[...].astype(jnp.bfloat16),
                        preferred_element_type=jnp.float32)
    q_ref[0] = jnp.dot(xf_ref[0].astype(jnp.bfloat16),
                       wq_ref[...].astype(jnp.bfloat16),
                       preferred_element_type=jnp.float32)


def _select_ge(keys, kk):
    rows = keys.shape[0]
    p = jnp.zeros((rows, 1), jnp.int32)
    for b in range(31, -1, -1):
        bit = np.int32(-2**31) if b == 31 else np.int32(1 << b)
        cand = p | bit
        thr = cand ^ _SIGN
        cnt = jnp.sum((keys >= thr).astype(jnp.int32), axis=1, keepdims=True)
        p = jnp.where(cnt >= kk, cand, p)
    return (keys >= (p ^ _SIGN)).astype(jnp.float32)


def _attn_body(q_ref, k_ref, v_ref, a1_ref, a2_ref, o_ref, *, kk1, kk2,
               scale):
    q = q_ref[0, 0]
    k = k_ref[0, 0]
    v = v_ref[0, 0]
    s = jax.lax.dot_general(q.astype(jnp.bfloat16), k.astype(jnp.bfloat16),
                            (((1,), (1,)), ((), ())),
                            preferred_element_type=jnp.float32) * scale
    keys = jax.lax.bitcast_convert_type(s, jnp.int32)
    keys = jnp.where(keys >= 0, keys, keys ^ np.int32(0x7FFFFFFF))
    m1 = _select_ge(keys, kk1)
    m2 = _select_ge(keys, kk2)
    m = jnp.max(s, axis=-1, keepdims=True)
    e = jnp.exp(s - m)
    s1 = jnp.sum(e * m1, axis=-1, keepdims=True)
    s2 = jnp.sum(e * m2, axis=-1, keepdims=True)
    a1 = a1_ref[0, 0]
    a2 = a2_ref[0, 0]
    w = e * (m1 * (a1 / s1) + m2 * (a2 / s2))
    o_ref[0, 0] = jnp.dot(w.astype(jnp.bfloat16), v.astype(jnp.bfloat16),
                          preferred_element_type=jnp.float32)


def _proj_body(x_ref, w_ref, b_ref, o_ref):
    o_ref[0] = jnp.dot(x_ref[0].astype(jnp.bfloat16),
                       w_ref[...].astype(jnp.bfloat16),
                       preferred_element_type=jnp.float32) + b_ref[...]


@jax.jit
def kernel(x, y, Wq, Wkv, Wproj, bproj, ln_w, ln_b, a1, a2):
    B, C, H, W = x.shape
    N = H * W
    hd = C // NUM_HEADS
    scale = hd ** (-0.5)
    kk1, kk2 = N // 2, N // 3

    pm = jnp.asarray(_pool_matrix(H))

    yT = y.reshape(B, C, N).transpose(0, 2, 1)
    xfT = x.reshape(B, C, N).transpose(0, 2, 1)

    kv, q = pl.pallas_call(
        _lnqkv_body,
        grid=(B,),
        in_specs=[
            pl.BlockSpec((1, N, C), lambda b: (b, 0, 0)),
            pl.BlockSpec((1, N, C), lambda b: (b, 0, 0)),
            pl.BlockSpec((N, N), lambda b: (0, 0)),
            pl.BlockSpec((C, 2 * C), lambda b: (0, 0)),
            pl.BlockSpec((C, C), lambda b: (0, 0)),
            pl.BlockSpec((1, C), lambda b: (0, 0)),
            pl.BlockSpec((1, C), lambda b: (0, 0)),
        ],
        out_specs=[
            pl.BlockSpec((1, N, 2 * C), lambda b: (b, 0, 0)),
            pl.BlockSpec((1, N, C), lambda b: (b, 0, 0)),
        ],
        out_shape=[
            jax.ShapeDtypeStruct((B, N, 2 * C), jnp.float32),
            jax.ShapeDtypeStruct((B, N, C), jnp.float32),
        ],
    )(yT, xfT, pm, Wkv, Wq, ln_w.reshape(1, C), ln_b.reshape(1, C))

    kh = kv[:, :, :C].reshape(B, N, NUM_HEADS, hd).transpose(0, 2, 1, 3)
    vh = kv[:, :, C:].reshape(B, N, NUM_HEADS, hd).transpose(0, 2, 1, 3)
    qh = q.reshape(B, N, NUM_HEADS, hd).transpose(0, 2, 1, 3)

    oh = pl.pallas_call(
        functools.partial(_attn_body, kk1=kk1, kk2=kk2, scale=scale),
        grid=(B, NUM_HEADS),
        in_specs=[
            pl.BlockSpec((1, 1, N, hd), lambda b, h: (b, h, 0, 0)),
            pl.BlockSpec((1, 1, N, hd), lambda b, h: (b, h, 0, 0)),
            pl.BlockSpec((1, 1, N, hd), lambda b, h: (b, h, 0, 0)),
            pl.BlockSpec((1, 1), lambda b, h: (0, 0)),
            pl.BlockSpec((1, 1), lambda b, h: (0, 0)),
        ],
        out_specs=pl.BlockSpec((1, 1, N, hd), lambda b, h: (b, h, 0, 0)),
        out_shape=jax.ShapeDtypeStruct((B, NUM_HEADS, N, hd), jnp.float32),
    )(qh, kh, vh, a1.reshape(1, 1), a2.reshape(1, 1))

    out = oh.transpose(0, 2, 1, 3).reshape(B, N, C)

    res = pl.pallas_call(
        _proj_body,
        grid=(B,),
        in_specs=[
            pl.BlockSpec((1, N, C), lambda b: (b, 0, 0)),
            pl.BlockSpec((C, C), lambda b: (0, 0)),
            pl.BlockSpec((1, C), lambda b: (0, 0)),
        ],
        out_specs=pl.BlockSpec((1, N, C), lambda b: (b, 0, 0)),
        out_shape=jax.ShapeDtypeStruct((B, N, C), jnp.float32),
    )(out, Wproj, bproj.reshape(1, C))

    return res.transpose(0, 2, 1).reshape(B, C, H, W)


# speedup vs baseline: 71.3245x; 1.5989x over previous
import functools

import jax
import jax.numpy as jnp
import numpy as np
from jax.experimental import pallas as pl

NUM_HEADS = 8
HDP = 128
_SIGN = np.int32(-2**31)


def _pool_matrix(hw: int) -> np.ndarray:
    p = np.zeros((hw * hw, hw * hw), np.float32)
    idx = np.arange(hw)
    for k in (3, 5, 7):
        a = (np.abs(idx[:, None] - idx[None, :]) <= k // 2).astype(np.float32)
        p += np.kron(a, a) / float(k * k)
    return p


def _stage1_body(y_ref, x_ref, pm_ref, wkT_ref, wvT_ref, wqT_ref,
                 lnw_ref, lnb_ref, kT_ref, vT_ref, qT_ref):
    ysT = jnp.dot(y_ref[0], pm_ref[...],
                  preferred_element_type=jnp.float32,
                  precision=jax.lax.Precision.HIGHEST)
    mu = jnp.mean(ysT, axis=0, keepdims=True)
    var = jnp.mean((ysT - mu) ** 2, axis=0, keepdims=True)
    ynT = (ysT - mu) / jnp.sqrt(var + 1e-5) * lnw_ref[...] + lnb_ref[...]
    ynT = ynT.astype(jnp.bfloat16)
    xT = x_ref[0].astype(jnp.bfloat16)
    kT_ref[0] = jnp.dot(wkT_ref[...], ynT, preferred_element_type=jnp.float32)
    vT_ref[0] = jnp.dot(wvT_ref[...], ynT, preferred_element_type=jnp.float32)
    qT_ref[0] = jnp.dot(wqT_ref[...], xT, preferred_element_type=jnp.float32)


def _select_ge(keys, kk):
    cols = keys.shape[1]
    p = jnp.zeros((1, cols), jnp.int32)
    for b in range(31, -1, -1):
        bit = np.int32(-2**31) if b == 31 else np.int32(1 << b)
        cand = p | bit
        thr = cand ^ _SIGN
        cnt = jnp.sum((keys >= thr).astype(jnp.int32), axis=0, keepdims=True)
        p = jnp.where(cnt >= kk, cand, p)
    return (keys >= (p ^ _SIGN)).astype(jnp.float32)


def _attn_body(qT_ref, kT_ref, vT_ref, wpT_ref, bp_ref, a1_ref, a2_ref,
               res_ref, *, kk1, kk2, scale):
    h = pl.program_id(1)
    kT = kT_ref[0].astype(jnp.bfloat16)
    qT = qT_ref[0].astype(jnp.bfloat16)
    sT = jax.lax.dot_general(kT, qT, (((0,), (0,)), ((), ())),
                             preferred_element_type=jnp.float32) * scale
    keys = jax.lax.bitcast_convert_type(sT, jnp.int32)
    keys = jnp.where(keys >= 0, keys, keys ^ np.int32(0x7FFFFFFF))
    m1 = _select_ge(keys, kk1)
    m2 = _select_ge(keys, kk2)
    m = jnp.max(sT, axis=0, keepdims=True)
    e = jnp.exp(sT - m)
    s1 = jnp.sum(e * m1, axis=0, keepdims=True)
    s2 = jnp.sum(e * m2, axis=0, keepdims=True)
    a1 = a1_ref[0, 0]
    a2 = a2_ref[0, 0]
    w = e * (m1 * (a1 / s1) + m2 * (a2 / s2))
    outT = jnp.dot(vT_ref[0].astype(jnp.bfloat16), w.astype(jnp.bfloat16),
                   preferred_element_type=jnp.float32)
    contrib = jnp.dot(wpT_ref[0].astype(jnp.bfloat16),
                      outT.astype(jnp.bfloat16),
                      preferred_element_type=jnp.float32)

    @pl.when(h == 0)
    def _init():
        res_ref[0] = contrib + bp_ref[...]

    @pl.when(h != 0)
    def _acc():
        res_ref[0] = res_ref[0] + contrib


@jax.jit
def kernel(x, y, Wq, Wkv, Wproj, bproj, ln_w, ln_b, a1, a2):
    B, C, H, W = x.shape
    N = H * W
    hd = C // NUM_HEADS
    scale = hd ** (-0.5)
    kk1, kk2 = N // 2, N // 3

    pm = jnp.asarray(_pool_matrix(H))

    y_flat = y.reshape(B, C, N)
    x_flat = x.reshape(B, C, N)

    def _headT(wmat):
        wt = wmat.reshape(C, NUM_HEADS, hd).transpose(1, 2, 0)
        wt = jnp.pad(wt, ((0, 0), (0, HDP - hd), (0, 0)))
        return wt.reshape(NUM_HEADS * HDP, C)

    wkT = _headT(Wkv[:, :C])
    wvT = _headT(Wkv[:, C:])
    wqT = _headT(Wq)
    wpT = jnp.pad(Wproj.reshape(NUM_HEADS, hd, C),
                  ((0, 0), (0, HDP - hd), (0, 0))).transpose(0, 2, 1)

    kT, vT, qT = pl.pallas_call(
        _stage1_body,
        grid=(B,),
        in_specs=[
            pl.BlockSpec((1, C, N), lambda b: (b, 0, 0)),
            pl.BlockSpec((1, C, N), lambda b: (b, 0, 0)),
            pl.BlockSpec((N, N), lambda b: (0, 0)),
            pl.BlockSpec((NUM_HEADS * HDP, C), lambda b: (0, 0)),
            pl.BlockSpec((NUM_HEADS * HDP, C), lambda b: (0, 0)),
            pl.BlockSpec((NUM_HEADS * HDP, C), lambda b: (0, 0)),
            pl.BlockSpec((C, 1), lambda b: (0, 0)),
            pl.BlockSpec((C, 1), lambda b: (0, 0)),
        ],
        out_specs=[
            pl.BlockSpec((1, NUM_HEADS * HDP, N), lambda b: (b, 0, 0)),
            pl.BlockSpec((1, NUM_HEADS * HDP, N), lambda b: (b, 0, 0)),
            pl.BlockSpec((1, NUM_HEADS * HDP, N), lambda b: (b, 0, 0)),
        ],
        out_shape=[
            jax.ShapeDtypeStruct((B, NUM_HEADS * HDP, N), jnp.float32),
            jax.ShapeDtypeStruct((B, NUM_HEADS * HDP, N), jnp.float32),
            jax.ShapeDtypeStruct((B, NUM_HEADS * HDP, N), jnp.float32),
        ],
    )(y_flat, x_flat, pm, wkT, wvT, wqT,
      ln_w.reshape(C, 1), ln_b.reshape(C, 1))

    resT = pl.pallas_call(
        functools.partial(_attn_body, kk1=kk1, kk2=kk2, scale=scale),
        grid=(B, NUM_HEADS),
        in_specs=[
            pl.BlockSpec((1, HDP, N), lambda b, h: (b, h, 0)),
            pl.BlockSpec((1, HDP, N), lambda b, h: (b, h, 0)),
            pl.BlockSpec((1, HDP, N), lambda b, h: (b, h, 0)),
            pl.BlockSpec((1, C, HDP), lambda b, h: (h, 0, 0)),
            pl.BlockSpec((C, 1), lambda b, h: (0, 0)),
            pl.BlockSpec((1, 1), lambda b, h: (0, 0)),
            pl.BlockSpec((1, 1), lambda b, h: (0, 0)),
        ],
        out_specs=pl.BlockSpec((1, C, N), lambda b, h: (b, 0, 0)),
        out_shape=jax.ShapeDtypeStruct((B, C, N), jnp.float32),
    )(qT, kT, vT, wpT, bproj.reshape(C, 1),
      a1.reshape(1, 1), a2.reshape(1, 1))

    return resT.reshape(B, C, H, W)


# speedup vs baseline: 116.6495x; 1.6355x over previous
import functools

import jax
import jax.numpy as jnp
import numpy as np
from jax.experimental import pallas as pl

NUM_HEADS = 8
HDP = 128
_SIGN = np.int32(-2**31)


def _pool_matrix(hw: int) -> np.ndarray:
    p = np.zeros((hw * hw, hw * hw), np.float32)
    idx = np.arange(hw)
    for k in (3, 5, 7):
        a = (np.abs(idx[:, None] - idx[None, :]) <= k // 2).astype(np.float32)
        p += np.kron(a, a) / float(k * k)
    return p


def _stage1_body(y_ref, x_ref, pm_ref, wkT_ref, wvT_ref, wqT_ref,
                 lnw_ref, lnb_ref, kT_ref, vT_ref, qT_ref):
    yb = y_ref[0]
    y_hi = yb.astype(jnp.bfloat16)
    y_lo = (yb - y_hi.astype(jnp.float32)).astype(jnp.bfloat16)
    pmat = pm_ref[...]
    p_hi = pmat.astype(jnp.bfloat16)
    p_lo = (pmat - p_hi.astype(jnp.float32)).astype(jnp.bfloat16)
    ysT = (jnp.dot(y_hi, p_hi, preferred_element_type=jnp.float32)
           + jnp.dot(y_hi, p_lo, preferred_element_type=jnp.float32)
           + jnp.dot(y_lo, p_hi, preferred_element_type=jnp.float32))
    mu = jnp.mean(ysT, axis=0, keepdims=True)
    var = jnp.mean((ysT - mu) ** 2, axis=0, keepdims=True)
    ynT = (ysT - mu) / jnp.sqrt(var + 1e-5) * lnw_ref[...] + lnb_ref[...]
    ynT = ynT.astype(jnp.bfloat16)
    xT = x_ref[0].astype(jnp.bfloat16)
    kT_ref[0] = jnp.dot(wkT_ref[...], ynT, preferred_element_type=jnp.float32)
    vT_ref[0] = jnp.dot(wvT_ref[...], ynT, preferred_element_type=jnp.float32)
    qT_ref[0] = jnp.dot(wqT_ref[...], xT, preferred_element_type=jnp.float32)


def _select_thr(keys, k16, kk):
    rows, cols = keys.shape
    p16 = jnp.zeros((1, cols), jnp.int16)
    negkk = np.int16(-kk)
    for b in range(14, -1, -1):
        cand = p16 | np.int16(1 << b)
        d = jnp.where(k16 >= cand, np.int16(-1), np.int16(0))
        h = rows
        while h > 16:
            h //= 2
            d = d[:h] + d[h:2 * h]
        s = jnp.sum(d.astype(jnp.int32), axis=0, keepdims=True)
        p16 = jnp.where(s.astype(jnp.int16) <= negkk, cand, p16)
    p = p16.astype(jnp.int32) << 15
    for b in range(14, 9, -1):
        cand = p | np.int32(1 << b)
        cnt = jnp.sum((keys >= cand).astype(jnp.int32), axis=0, keepdims=True)
        p = jnp.where(cnt >= kk, cand, p)
    return p


def _attn_body(qT_ref, kT_ref, vT_ref, wpT_ref, bp_ref, a1_ref, a2_ref,
               res_ref, *, kk1, kk2, scale):
    h = pl.program_id(1)
    kT = kT_ref[0].astype(jnp.bfloat16)
    qT = qT_ref[0].astype(jnp.bfloat16)
    sT = jax.lax.dot_general(kT, qT, (((0,), (0,)), ((), ())),
                             preferred_element_type=jnp.float32) * scale
    m = jnp.max(sT, axis=0, keepdims=True)
    e = jnp.exp(sT - m)
    keys = jax.lax.bitcast_convert_type(e, jnp.int32)
    k16 = (keys >> 15).astype(jnp.int16)
    p1 = _select_thr(keys, k16, kk1)
    p2 = _select_thr(keys, k16, kk2)
    m1 = keys >= p1
    m2 = keys >= p2
    zero = jnp.float32(0.0)
    s1 = jnp.sum(jnp.where(m1, e, zero), axis=0, keepdims=True)
    s2 = jnp.sum(jnp.where(m2, e, zero), axis=0, keepdims=True)
    a1 = a1_ref[0, 0]
    a2 = a2_ref[0, 0]
    w = jnp.where(m1, e * (a1 / s1), zero) + jnp.where(m2, e * (a2 / s2),
                                                       zero)
    outT = jnp.dot(vT_ref[0].astype(jnp.bfloat16), w.astype(jnp.bfloat16),
                   preferred_element_type=jnp.float32)
    contrib = jnp.dot(wpT_ref[0].astype(jnp.bfloat16),
                      outT.astype(jnp.bfloat16),
                      preferred_element_type=jnp.float32)

    @pl.when(h == 0)
    def _init():
        res_ref[0] = contrib + bp_ref[...]

    @pl.when(h != 0)
    def _acc():
        res_ref[0] = res_ref[0] + contrib


@jax.jit
def kernel(x, y, Wq, Wkv, Wproj, bproj, ln_w, ln_b, a1, a2):
    B, C, H, W = x.shape
    N = H * W
    hd = C // NUM_HEADS
    scale = hd ** (-0.5)
    kk1, kk2 = N // 2, N // 3

    pm = jnp.asarray(_pool_matrix(H))

    y_flat = y.reshape(B, C, N)
    x_flat = x.reshape(B, C, N)

    def _headT(wmat):
        wt = wmat.reshape(C, NUM_HEADS, hd).transpose(1, 2, 0)
        wt = jnp.pad(wt, ((0, 0), (0, HDP - hd), (0, 0)))
        return wt.reshape(NUM_HEADS * HDP, C)

    wkT = _headT(Wkv[:, :C])
    wvT = _headT(Wkv[:, C:])
    wqT = _headT(Wq)
    wpT = jnp.pad(Wproj.reshape(NUM_HEADS, hd, C),
                  ((0, 0), (0, HDP - hd), (0, 0))).transpose(0, 2, 1)

    kT, vT, qT = pl.pallas_call(
        _stage1_body,
        grid=(B,),
        in_specs=[
            pl.BlockSpec((1, C, N), lambda b: (b, 0, 0)),
            pl.BlockSpec((1, C, N), lambda b: (b, 0, 0)),
            pl.BlockSpec((N, N), lambda b: (0, 0)),
            pl.BlockSpec((NUM_HEADS * HDP, C), lambda b: (0, 0)),
            pl.BlockSpec((NUM_HEADS * HDP, C), lambda b: (0, 0)),
            pl.BlockSpec((NUM_HEADS * HDP, C), lambda b: (0, 0)),
            pl.BlockSpec((C, 1), lambda b: (0, 0)),
            pl.BlockSpec((C, 1), lambda b: (0, 0)),
        ],
        out_specs=[
            pl.BlockSpec((1, NUM_HEADS * HDP, N), lambda b: (b, 0, 0)),
            pl.BlockSpec((1, NUM_HEADS * HDP, N), lambda b: (b, 0, 0)),
            pl.BlockSpec((1, NUM_HEADS * HDP, N), lambda b: (b, 0, 0)),
        ],
        out_shape=[
            jax.ShapeDtypeStruct((B, NUM_HEADS * HDP, N), jnp.float32),
            jax.ShapeDtypeStruct((B, NUM_HEADS * HDP, N), jnp.float32),
            jax.ShapeDtypeStruct((B, NUM_HEADS * HDP, N), jnp.float32),
        ],
    )(y_flat, x_flat, pm, wkT, wvT, wqT,
      ln_w.reshape(C, 1), ln_b.reshape(C, 1))

    resT = pl.pallas_call(
        functools.partial(_attn_body, kk1=kk1, kk2=kk2, scale=scale),
        grid=(B, NUM_HEADS),
        in_specs=[
            pl.BlockSpec((1, HDP, N), lambda b, h: (b, h, 0)),
            pl.BlockSpec((1, HDP, N), lambda b, h: (b, h, 0)),
            pl.BlockSpec((1, HDP, N), lambda b, h: (b, h, 0)),
            pl.BlockSpec((1, C, HDP), lambda b, h: (h, 0, 0)),
            pl.BlockSpec((C, 1), lambda b, h: (0, 0)),
            pl.BlockSpec((1, 1), lambda b, h: (0, 0)),
            pl.BlockSpec((1, 1), lambda b, h: (0, 0)),
        ],
        out_specs=pl.BlockSpec((1, C, N), lambda b, h: (b, 0, 0)),
        out_shape=jax.ShapeDtypeStruct((B, C, N), jnp.float32),
    )(qT, kT, vT, wpT, bproj.reshape(C, 1),
      a1.reshape(1, 1), a2.reshape(1, 1))

    return resT.reshape(B, C, H, W)
